# Initial kernel scaffold; baseline (speedup 1.0000x reference)
import jax
import jax.numpy as jnp
from jax.experimental import pallas as pl


def kernel(x, source, position_ids, W, r, window_size):
    raise NotImplementedError("write your pallas kernel here")


# speedup vs baseline: 27.3062x; 27.3062x over previous
import functools

import jax
import jax.numpy as jnp
from jax import lax
from jax.experimental import pallas as pl
from jax.experimental.pallas import tpu as pltpu
from jax.experimental.pallas import tpu_sc as plsc

_B, _S, _D = 4, 2048, 768
_G = 64
_R = 512
_KEEP = _S - _R
_NC, _NS = 2, 16
_NW = _NC * _NS
_TOT = _B * _KEEP
_RPW = _TOT // _NW
_CH = 16
_NCH = _RPW // _CH


def _score_body(x_ref, w_ref, rank_ref, norm_ref, simrow_ref):
    xb = x_ref[0]
    w = w_ref[...]
    g = lax.dot_general(xb, w, (((1,), (1,)), ((), ())),
                        preferred_element_type=jnp.float32)
    gn = jnp.sqrt(jnp.sum(g * g, axis=1, keepdims=True))
    gnorm = g / jnp.maximum(gn, 1e-12)
    shifted = jnp.concatenate(
        [gnorm[1:], jnp.zeros((1, _G), jnp.float32)], axis=0)
    sim = jnp.sum(gnorm * shifted, axis=1, keepdims=True)
    pid = lax.broadcasted_iota(jnp.int32, (_S, 1), 0)
    sim = jnp.where(pid >= _S - 1, jnp.float32(-2.0), sim)
    simrow_ref[...] = jnp.reshape(sim, (1, _S))
    qi0 = lax.broadcasted_iota(jnp.int32, (_S, 128), 1)

    def chunk(c, acc):
        qs = simrow_ref[:, pl.ds(c * 128, 128)]
        qi = qi0 + c * 128
        gt = qs > sim
        eq = (qs == sim) & (qi < pid)
        return acc + (gt | eq).astype(jnp.int32)

    acc = lax.fori_loop(0, _S // 128, chunk, jnp.zeros((_S, 128), jnp.int32))
    rank = jnp.sum(acc, axis=1, keepdims=True)
    rank_ref[...] = jnp.reshape(rank, (1, 1, _S))
    norm_ref[...] = jnp.reshape(gn, (1, 1, _S))


def _scores(x, w):
    return pl.pallas_call(
        _score_body,
        grid=(_B,),
        in_specs=[pl.BlockSpec((1, _S, _D), lambda b: (b, 0, 0)),
                  pl.BlockSpec((_G, _D), lambda b: (0, 0))],
        out_specs=[pl.BlockSpec((1, 1, _S), lambda b: (b, 0, 0)),
                   pl.BlockSpec((1, 1, _S), lambda b: (b, 0, 0))],
        out_shape=[jax.ShapeDtypeStruct((_B, 1, _S), jnp.int32),
                   jax.ShapeDtypeStruct((_B, 1, _S), jnp.float32)],
        scratch_shapes=[pltpu.VMEM((1, _S), jnp.float32)],
    )(x, w)


def _greedy_body(rank_hbm, norm_hbm, pos_hbm,
                 kg_hbm, kj_hbm, c0_hbm, c1_hbm, m_hbm, po_hbm,
                 rank_v, order_v, used_v, isi_v, norms_v, posrow_v,
                 k_v, kj_v, c0_v, c1_v, m_v, pos_v):
    wid = lax.axis_index("s") * _NC + lax.axis_index("c")

    @pl.when(wid < _B)
    def _():
        b = wid
        pltpu.sync_copy(rank_hbm.at[b], rank_v)
        pltpu.sync_copy(norm_hbm.at[b], norms_v.at[pl.ds(0, _S)])
        pltpu.sync_copy(pos_hbm.at[b], posrow_v)
        norms_v[pl.ds(_S, 16)] = jnp.zeros((16,), jnp.float32)

        zeros16 = jnp.zeros((16,), jnp.int32)

        def init_chunk(ci, _):
            b16 = ci * 16
            used_v[pl.ds(b16, 16)] = zeros16
            isi_v[pl.ds(b16, 16)] = zeros16
            vals = lax.iota(jnp.int32, 16) + b16
            idx = rank_v[pl.ds(b16, 16)]
            plsc.store_scatter(order_v, [idx], vals)
            return 0

        lax.fori_loop(0, _S // 16, init_chunk, 0)

        lanes = lax.iota(jnp.int32, 16)
        ones16 = jnp.ones((16,), jnp.int32)

        def g_cond(carry):
            t, cnt = carry
            return jnp.logical_and(t < _S - 1, cnt < _R)

        def g_body(carry):
            t, cnt = carry
            p = order_v[pl.ds(t, 16)][0]
            u2 = used_v[pl.ds(p, 16)]
            free = (u2[0] + u2[1]) == 0

            @pl.when(free)
            def _():
                plsc.store_scatter(used_v, [p + lanes], ones16,
                                   mask=lanes < 2)
                plsc.store_scatter(isi_v, [p + lanes], ones16,
                                   mask=lanes < 1)

            return (t + 1, cnt + free.astype(jnp.int32))

        lax.while_loop(g_cond, g_body, (jnp.int32(0), jnp.int32(0)))

        def comp_chunk(ci, base):
            b16 = ci * 16
            tvec = lax.iota(jnp.int32, 16) + b16
            usedc = used_v[pl.ds(b16, 16)]
            isic = isi_v[pl.ds(b16, 16)]
            keep = jnp.logical_or(usedc == 0, isic == 1)
            kint = keep.astype(jnp.int32)
            incl = plsc.cumsum(kint)
            posv = base + incl - kint
            n0 = norms_v[pl.ds(b16, 16)]
            n1 = norms_v[pl.ds(b16 + 1, 16)]
            isb = isic == 1
            den = n0 + n1 + jnp.float32(1e-8)
            c0 = jnp.where(isb, n0 / den, jnp.float32(1.0))
            c1 = jnp.where(isb, n1 / den, jnp.float32(0.0))
            mm = jnp.where(isb, jnp.float32(1.0), jnp.float32(0.0))
            gbase = b * _S
            plsc.store_scatter(k_v, [posv], tvec + gbase, mask=keep)
            plsc.store_scatter(kj_v, [posv], tvec + isic + gbase, mask=keep)
            plsc.store_scatter(c0_v, [posv], c0, mask=keep)
            plsc.store_scatter(c1_v, [posv], c1, mask=keep)
            plsc.store_scatter(m_v, [posv], mm, mask=keep)
            plsc.store_scatter(pos_v, [posv], posrow_v[pl.ds(b16, 16)],
                               mask=keep)
            return base + jnp.sum(kint)

        lax.fori_loop(0, _S // 16, comp_chunk, jnp.int32(0))

        pltpu.sync_copy(k_v, kg_hbm.at[b])
        pltpu.sync_copy(kj_v, kj_hbm.at[b])
        pltpu.sync_copy(c0_v, c0_hbm.at[b])
        pltpu.sync_copy(c1_v, c1_hbm.at[b])
        pltpu.sync_copy(m_v, m_hbm.at[b])
        pltpu.sync_copy(pos_v, po_hbm.at[b])


def _greedy(rank, norm, position_ids):
    mesh = plsc.VectorSubcoreMesh(core_axis_name="c", subcore_axis_name="s",
                                  num_cores=_NC, num_subcores=_NS)
    f = pl.kernel(
        _greedy_body,
        out_type=[jax.ShapeDtypeStruct((_B, _KEEP), jnp.int32),
                  jax.ShapeDtypeStruct((_B, _KEEP), jnp.int32),
                  jax.ShapeDtypeStruct((_B, _KEEP), jnp.float32),
                  jax.ShapeDtypeStruct((_B, _KEEP), jnp.float32),
                  jax.ShapeDtypeStruct((_B, _KEEP), jnp.float32),
                  jax.ShapeDtypeStruct((_B, _KEEP), jnp.int32)],
        mesh=mesh,
        scratch_types=[pltpu.VMEM((_S,), jnp.int32),
                       pltpu.VMEM((_S + 16,), jnp.int32),
                       pltpu.VMEM((_S + 16,), jnp.int32),
                       pltpu.VMEM((_S + 16,), jnp.int32),
                       pltpu.VMEM((_S + 16,), jnp.float32),
                       pltpu.VMEM((_S,), jnp.int32),
                       pltpu.VMEM((_KEEP,), jnp.int32),
                       pltpu.VMEM((_KEEP,), jnp.int32),
                       pltpu.VMEM((_KEEP,), jnp.float32),
                       pltpu.VMEM((_KEEP,), jnp.float32),
                       pltpu.VMEM((_KEEP,), jnp.float32),
                       pltpu.VMEM((_KEEP,), jnp.int32)],
        compiler_params=pltpu.CompilerParams(needs_layout_passes=False),
    )
    return f(rank, norm, position_ids)


def _merge_body(x_hbm, s_hbm, kg_hbm, kj_hbm, c0_hbm, c1_hbm, m_hbm,
                xo_hbm, so_hbm,
                kg_v, kj_v, c0_v, c1_v, m_v, xa_v, xb_v, sa_v, sb_v,
                sem_a, sem_b):
    wid = lax.axis_index("s") * _NC + lax.axis_index("c")
    base = wid * _RPW
    pltpu.sync_copy(kg_hbm.at[pl.ds(base, _RPW)], kg_v)
    pltpu.sync_copy(kj_hbm.at[pl.ds(base, _RPW)], kj_v)
    pltpu.sync_copy(c0_hbm.at[pl.ds(base, _RPW)], c0_v.at[pl.ds(0, _RPW)])
    pltpu.sync_copy(c1_hbm.at[pl.ds(base, _RPW)], c1_v.at[pl.ds(0, _RPW)])
    pltpu.sync_copy(m_hbm.at[pl.ds(base, _RPW)], m_v.at[pl.ds(0, _RPW)])

    for c in range(_NCH):
        cb = c * _CH
        ia = kg_v[pl.ds(cb, _CH)]
        ib = kj_v[pl.ds(cb, _CH)]
        cpa = pltpu.async_copy(x_hbm.at[ia], xa_v, sem_a)
        cpb = pltpu.async_copy(x_hbm.at[ib], xb_v, sem_b)
        cpa.wait()
        cpb.wait()

        def xrow(o, _):
            ms = m_v[pl.ds(cb + o, 16)][0]

            @pl.when(ms > 0.5)
            def _():
                c0s = c0_v[pl.ds(cb + o, 16)][0]
                c1s = c1_v[pl.ds(cb + o, 16)][0]

                def xlane(d, _):
                    sl = pl.ds(d * 16, 16)
                    xa_v[o, sl] = c0s * xa_v[o, sl] + c1s * xb_v[o, sl]
                    return 0

                lax.fori_loop(0, _D // 16, xlane, 0)

            return 0

        lax.fori_loop(0, _CH, xrow, 0)
        pltpu.sync_copy(xa_v, xo_hbm.at[pl.ds(base + cb, _CH)])

    for c in range(_NCH):
        cb = c * _CH
        ia = kg_v[pl.ds(cb, _CH)]
        ib = kj_v[pl.ds(cb, _CH)]
        cpa = pltpu.async_copy(s_hbm.at[ia], sa_v, sem_a)
        cpb = pltpu.async_copy(s_hbm.at[ib], sb_v, sem_b)
        cpa.wait()
        cpb.wait()

        def srow(o, _):
            ms = m_v[pl.ds(cb + o, 16)][0]

            @pl.when(ms > 0.5)
            def _():
                def slane(d, _):
                    sl = pl.ds(d * 16, 16)
                    sa_v[o, sl] = sa_v[o, sl] + sb_v[o, sl]
                    return 0

                lax.fori_loop(0, _S // 16, slane, 0)

            return 0

        lax.fori_loop(0, _CH, srow, 0)
        pltpu.sync_copy(sa_v, so_hbm.at[pl.ds(base + cb, _CH)])


def _merge(x2, s2, kg, kj, c0, c1, m):
    mesh = plsc.VectorSubcoreMesh(core_axis_name="c", subcore_axis_name="s",
                                  num_cores=_NC, num_subcores=_NS)
    f = pl.kernel(
        _merge_body,
        out_type=[jax.ShapeDtypeStruct((_TOT, _D), jnp.float32),
                  jax.ShapeDtypeStruct((_TOT, _S), jnp.float32)],
        mesh=mesh,
        scratch_types=[pltpu.VMEM((_RPW,), jnp.int32),
                       pltpu.VMEM((_RPW,), jnp.int32),
                       pltpu.VMEM((_RPW + 16,), jnp.float32),
                       pltpu.VMEM((_RPW + 16,), jnp.float32),
                       pltpu.VMEM((_RPW + 16,), jnp.float32),
                       pltpu.VMEM((_CH, _D), jnp.float32),
                       pltpu.VMEM((_CH, _D), jnp.float32),
                       pltpu.VMEM((_CH, _S), jnp.float32),
                       pltpu.VMEM((_CH, _S), jnp.float32),
                       pltpu.SemaphoreType.DMA,
                       pltpu.SemaphoreType.DMA],
        compiler_params=pltpu.CompilerParams(needs_layout_passes=False),
    )
    return f(x2, s2, kg, kj, c0, c1, m)


def kernel(x, source, position_ids, W, r, window_size):
    anchor = (jnp.asarray(r) - _R) + (jnp.asarray(window_size) - 2)
    x = x + anchor.astype(x.dtype) * 0
    rank, norm = _scores(x, W)
    rank = rank.reshape(_B, _S)
    norm = norm.reshape(_B, _S)
    kg, kj, c0, c1, m, pos_out = _greedy(rank, norm, position_ids)
    x2 = x.reshape(_B * _S, _D)
    s2 = source.reshape(_B * _S, _S)
    xo, so = _merge(x2, s2, kg.reshape(_TOT), kj.reshape(_TOT),
                    c0.reshape(_TOT), c1.reshape(_TOT), m.reshape(_TOT))
    return (xo.reshape(_B, _KEEP, _D), so.reshape(_B, _KEEP, _S), pos_out)


# speedup vs baseline: 30.1651x; 1.1047x over previous
import functools

import jax
import jax.numpy as jnp
from jax import lax
from jax.experimental import pallas as pl
from jax.experimental.pallas import tpu as pltpu
from jax.experimental.pallas import tpu_sc as plsc

_B, _S, _D = 4, 2048, 768
_G = 64
_R = 512
_KEEP = _S - _R
_NC, _NS = 2, 16
_NW = _NC * _NS
_TOT = _B * _KEEP
_UN = _KEEP - _R
_UTOT = _B * _UN
_MTOT = _B * _R
_UPW = _UTOT // _NW
_MPW = _MTOT // _NW
_CH = 16


def _score_body(x_ref, w_ref, rank_ref, norm_ref, simrow_ref):
    xb = x_ref[0]
    w = w_ref[...]
    g = lax.dot_general(xb, w, (((1,), (1,)), ((), ())),
                        preferred_element_type=jnp.float32)
    gn = jnp.sqrt(jnp.sum(g * g, axis=1, keepdims=True))
    gnorm = g / jnp.maximum(gn, 1e-12)
    shifted = jnp.concatenate(
        [gnorm[1:], jnp.zeros((1, _G), jnp.float32)], axis=0)
    sim = jnp.sum(gnorm * shifted, axis=1, keepdims=True)
    pid = lax.broadcasted_iota(jnp.int32, (_S, 1), 0)
    sim = jnp.where(pid >= _S - 1, jnp.float32(-2.0), sim)
    simrow_ref[...] = jnp.reshape(sim, (1, _S))
    qi0 = lax.broadcasted_iota(jnp.int32, (_S, 128), 1)

    def chunk(c, acc):
        qs = simrow_ref[:, pl.ds(c * 128, 128)]
        qi = qi0 + c * 128
        gt = qs > sim
        eq = (qs == sim) & (qi < pid)
        return acc + (gt | eq).astype(jnp.int32)

    acc = lax.fori_loop(0, _S // 128, chunk, jnp.zeros((_S, 128), jnp.int32))
    rank = jnp.sum(acc, axis=1, keepdims=True)
    rank_ref[...] = jnp.reshape(rank, (1, 1, _S))
    norm_ref[...] = jnp.reshape(gn, (1, 1, _S))


def _scores(x, w):
    return pl.pallas_call(
        _score_body,
        grid=(_B,),
        in_specs=[pl.BlockSpec((1, _S, _D), lambda b: (b, 0, 0)),
                  pl.BlockSpec((_G, _D), lambda b: (0, 0))],
        out_specs=[pl.BlockSpec((1, 1, _S), lambda b: (b, 0, 0)),
                   pl.BlockSpec((1, 1, _S), lambda b: (b, 0, 0))],
        out_shape=[jax.ShapeDtypeStruct((_B, 1, _S), jnp.int32),
                   jax.ShapeDtypeStruct((_B, 1, _S), jnp.float32)],
        scratch_shapes=[pltpu.VMEM((1, _S), jnp.float32)],
    )(x, w)


def _greedy_body(rank_hbm, norm_hbm, pos_hbm,
                 ui_hbm, up_hbm, mi_hbm, mp_hbm, c0_hbm, c1_hbm, po_hbm,
                 rank_v, order_v, used_v, isi_v, norms_v, posrow_v,
                 ui_v, up_v, mi_v, mp_v, c0_v, c1_v, pos_v):
    wid = lax.axis_index("s") * _NC + lax.axis_index("c")

    @pl.when(wid < _B)
    def _():
        b = wid
        pltpu.sync_copy(rank_hbm.at[b], rank_v)
        pltpu.sync_copy(norm_hbm.at[b], norms_v.at[pl.ds(0, _S)])
        pltpu.sync_copy(pos_hbm.at[b], posrow_v)
        norms_v[pl.ds(_S, 16)] = jnp.zeros((16,), jnp.float32)

        zeros16 = jnp.zeros((16,), jnp.int32)

        def init_chunk(ci, _):
            b16 = ci * 16
            used_v[pl.ds(b16, 16)] = zeros16
            isi_v[pl.ds(b16, 16)] = zeros16
            vals = lax.iota(jnp.int32, 16) + b16
            idx = rank_v[pl.ds(b16, 16)]
            plsc.store_scatter(order_v, [idx], vals)
            return 0

        lax.fori_loop(0, _S // 16, init_chunk, 0)

        lanes = lax.iota(jnp.int32, 16)
        ones16 = jnp.ones((16,), jnp.int32)

        def g_cond(carry):
            t, cnt = carry
            return jnp.logical_and(t < _S - 1, cnt < _R)

        def g_body(carry):
            t, cnt = carry
            p = order_v[pl.ds(t, 16)][0]
            u2 = used_v[pl.ds(p, 16)]
            free = (u2[0] + u2[1]) == 0

            @pl.when(free)
            def _():
                plsc.store_scatter(used_v, [p + lanes], ones16,
                                   mask=lanes < 2)
                plsc.store_scatter(isi_v, [p + lanes], ones16,
                                   mask=lanes < 1)

            return (t + 1, cnt + free.astype(jnp.int32))

        lax.while_loop(g_cond, g_body, (jnp.int32(0), jnp.int32(0)))

        def comp_chunk(ci, carry):
            base, ubase, mbase = carry
            b16 = ci * 16
            tvec = lax.iota(jnp.int32, 16) + b16
            usedc = used_v[pl.ds(b16, 16)]
            isic = isi_v[pl.ds(b16, 16)]
            isb = isic == 1
            keep = jnp.logical_or(usedc == 0, isb)
            kint = keep.astype(jnp.int32)
            posv = base + plsc.cumsum(kint) - kint
            ukeep = jnp.logical_and(keep, jnp.logical_not(isb))
            uint = ukeep.astype(jnp.int32)
            uposv = ubase + plsc.cumsum(uint) - uint
            mint = isic
            mposv = mbase + plsc.cumsum(mint) - mint
            n0 = norms_v[pl.ds(b16, 16)]
            n1 = norms_v[pl.ds(b16 + 1, 16)]
            den = n0 + n1 + jnp.float32(1e-8)
            gin = b * _S
            gout = b * _KEEP
            plsc.store_scatter(ui_v, [uposv], tvec + gin, mask=ukeep)
            plsc.store_scatter(up_v, [uposv], posv + gout, mask=ukeep)
            plsc.store_scatter(mi_v, [mposv], tvec + gin, mask=isb)
            plsc.store_scatter(mp_v, [mposv], posv + gout, mask=isb)
            plsc.store_scatter(c0_v, [mposv], n0 / den, mask=isb)
            plsc.store_scatter(c1_v, [mposv], n1 / den, mask=isb)
            plsc.store_scatter(pos_v, [posv], posrow_v[pl.ds(b16, 16)],
                               mask=keep)
            return (base + jnp.sum(kint), ubase + jnp.sum(uint),
                    mbase + jnp.sum(mint))

        lax.fori_loop(0, _S // 16, comp_chunk,
                      (jnp.int32(0), jnp.int32(0), jnp.int32(0)))

        pltpu.sync_copy(ui_v, ui_hbm.at[b])
        pltpu.sync_copy(up_v, up_hbm.at[b])
        pltpu.sync_copy(mi_v, mi_hbm.at[b])
        pltpu.sync_copy(mp_v, mp_hbm.at[b])
        pltpu.sync_copy(c0_v, c0_hbm.at[b])
        pltpu.sync_copy(c1_v, c1_hbm.at[b])
        pltpu.sync_copy(pos_v, po_hbm.at[b])


def _greedy(rank, norm, position_ids):
    mesh = plsc.VectorSubcoreMesh(core_axis_name="c", subcore_axis_name="s",
                                  num_cores=_NC, num_subcores=_NS)
    f = pl.kernel(
        _greedy_body,
        out_type=[jax.ShapeDtypeStruct((_B, _UN), jnp.int32),
                  jax.ShapeDtypeStruct((_B, _UN), jnp.int32),
                  jax.ShapeDtypeStruct((_B, _R), jnp.int32),
                  jax.ShapeDtypeStruct((_B, _R), jnp.int32),
                  jax.ShapeDtypeStruct((_B, _R), jnp.float32),
                  jax.ShapeDtypeStruct((_B, _R), jnp.float32),
                  jax.ShapeDtypeStruct((_B, _KEEP), jnp.int32)],
        mesh=mesh,
        scratch_types=[pltpu.VMEM((_S,), jnp.int32),
                       pltpu.VMEM((_S + 16,), jnp.int32),
                       pltpu.VMEM((_S + 16,), jnp.int32),
                       pltpu.VMEM((_S + 16,), jnp.int32),
                       pltpu.VMEM((_S + 16,), jnp.float32),
                       pltpu.VMEM((_S,), jnp.int32),
                       pltpu.VMEM((_UN,), jnp.int32),
                       pltpu.VMEM((_UN,), jnp.int32),
                       pltpu.VMEM((_R,), jnp.int32),
                       pltpu.VMEM((_R,), jnp.int32),
                       pltpu.VMEM((_R,), jnp.float32),
                       pltpu.VMEM((_R,), jnp.float32),
                       pltpu.VMEM((_KEEP,), jnp.int32)],
        compiler_params=pltpu.CompilerParams(needs_layout_passes=False),
    )
    return f(rank, norm, position_ids)


def _merge_body(x_hbm, s_hbm, ui_hbm, up_hbm, mi_hbm, mp_hbm, c0_hbm, c1_hbm,
                xo_hbm, so_hbm,
                ui_v, up_v, mi_v, mp_v, c0_v, c1_v, xa_v, xb_v, sa_v, sb_v,
                sem_a, sem_b):
    wid = lax.axis_index("s") * _NC + lax.axis_index("c")
    ub = wid * _UPW
    mb = wid * _MPW
    pltpu.sync_copy(ui_hbm.at[pl.ds(ub, _UPW)], ui_v)
    pltpu.sync_copy(up_hbm.at[pl.ds(ub, _UPW)], up_v)
    pltpu.sync_copy(mi_hbm.at[pl.ds(mb, _MPW)], mi_v)
    pltpu.sync_copy(mp_hbm.at[pl.ds(mb, _MPW)], mp_v)
    pltpu.sync_copy(c0_hbm.at[pl.ds(mb, _MPW)], c0_v.at[pl.ds(0, _MPW)])
    pltpu.sync_copy(c1_hbm.at[pl.ds(mb, _MPW)], c1_v.at[pl.ds(0, _MPW)])

    for c in range(_UPW // _CH):
        cb = c * _CH
        ia = ui_v[pl.ds(cb, _CH)]
        op = up_v[pl.ds(cb, _CH)]
        pltpu.async_copy(x_hbm.at[ia], xa_v, sem_a).wait()
        pltpu.async_copy(xa_v, xo_hbm.at[op], sem_a).wait()
        pltpu.async_copy(s_hbm.at[ia], sa_v, sem_b).wait()
        pltpu.async_copy(sa_v, so_hbm.at[op], sem_b).wait()

    for c in range(_MPW // _CH):
        cb = c * _CH
        ia = mi_v[pl.ds(cb, _CH)]
        ib = ia + 1
        op = mp_v[pl.ds(cb, _CH)]
        cpa = pltpu.async_copy(x_hbm.at[ia], xa_v, sem_a)
        cpb = pltpu.async_copy(x_hbm.at[ib], xb_v, sem_b)
        cpa.wait()
        cpb.wait()

        def xrow(o, _):
            c0s = c0_v[pl.ds(cb + o, 16)][0]
            c1s = c1_v[pl.ds(cb + o, 16)][0]

            def xlane(d, _):
                sl = pl.ds(d * 16, 16)
                xa_v[o, sl] = c0s * xa_v[o, sl] + c1s * xb_v[o, sl]
                return 0

            lax.fori_loop(0, _D // 16, xlane, 0)
            return 0

        lax.fori_loop(0, _CH, xrow, 0)
        pltpu.async_copy(xa_v, xo_hbm.at[op], sem_a).wait()

        cpa = pltpu.async_copy(s_hbm.at[ia], sa_v, sem_a)
        cpb = pltpu.async_copy(s_hbm.at[ib], sb_v, sem_b)
        cpa.wait()
        cpb.wait()

        def srow(o, _):
            def slane(d, _):
                sl = pl.ds(d * 16, 16)
                sa_v[o, sl] = sa_v[o, sl] + sb_v[o, sl]
                return 0

            lax.fori_loop(0, _S // 16, slane, 0)
            return 0

        lax.fori_loop(0, _CH, srow, 0)
        pltpu.async_copy(sa_v, so_hbm.at[op], sem_a).wait()


def _merge(x2, s2, ui, up, mi, mp, c0, c1):
    mesh = plsc.VectorSubcoreMesh(core_axis_name="c", subcore_axis_name="s",
                                  num_cores=_NC, num_subcores=_NS)
    f = pl.kernel(
        _merge_body,
        out_type=[jax.ShapeDtypeStruct((_TOT, _D), jnp.float32),
                  jax.ShapeDtypeStruct((_TOT, _S), jnp.float32)],
        mesh=mesh,
        scratch_types=[pltpu.VMEM((_UPW,), jnp.int32),
                       pltpu.VMEM((_UPW,), jnp.int32),
                       pltpu.VMEM((_MPW,), jnp.int32),
                       pltpu.VMEM((_MPW,), jnp.int32),
                       pltpu.VMEM((_MPW + 16,), jnp.float32),
                       pltpu.VMEM((_MPW + 16,), jnp.float32),
                       pltpu.VMEM((_CH, _D), jnp.float32),
                       pltpu.VMEM((_CH, _D), jnp.float32),
                       pltpu.VMEM((_CH, _S), jnp.float32),
                       pltpu.VMEM((_CH, _S), jnp.float32),
                       pltpu.SemaphoreType.DMA,
                       pltpu.SemaphoreType.DMA],
        compiler_params=pltpu.CompilerParams(needs_layout_passes=False),
    )
    return f(x2, s2, ui, up, mi, mp, c0, c1)


def kernel(x, source, position_ids, W, r, window_size):
    anchor = (jnp.asarray(r) - _R) + (jnp.asarray(window_size) - 2)
    x = x + anchor.astype(x.dtype) * 0
    rank, norm = _scores(x, W)
    rank = rank.reshape(_B, _S)
    norm = norm.reshape(_B, _S)
    ui, up, mi, mp, c0, c1, pos_out = _greedy(rank, norm, position_ids)
    x2 = x.reshape(_B * _S, _D)
    s2 = source.reshape(_B * _S, _S)
    xo, so = _merge(x2, s2, ui.reshape(_UTOT), up.reshape(_UTOT),
                    mi.reshape(_MTOT), mp.reshape(_MTOT),
                    c0.reshape(_MTOT), c1.reshape(_MTOT))
    return (xo.reshape(_B, _KEEP, _D), so.reshape(_B, _KEEP, _S), pos_out)


# speedup vs baseline: 32.9495x; 1.0923x over previous
import functools

import jax
import jax.numpy as jnp
from jax import lax
from jax.experimental import pallas as pl
from jax.experimental.pallas import tpu as pltpu
from jax.experimental.pallas import tpu_sc as plsc

_B, _S, _D = 4, 2048, 768
_G = 64
_R = 512
_KEEP = _S - _R
_NC, _NS = 2, 16
_NW = _NC * _NS
_TOT = _B * _KEEP
_UN = _KEEP - _R
_UTOT = _B * _UN
_MTOT = _B * _R
_UPW = _UTOT // _NW
_MPW = _MTOT // _NW
_CH = 8
_NU = _UPW // _CH
_NM = _MPW // _CH


def _score_body(x_ref, w_ref, rank_ref, norm_ref, simrow_ref):
    xb = x_ref[0]
    w = w_ref[...]
    g = lax.dot_general(xb, w, (((1,), (1,)), ((), ())),
                        preferred_element_type=jnp.float32)
    gn = jnp.sqrt(jnp.sum(g * g, axis=1, keepdims=True))
    gnorm = g / jnp.maximum(gn, 1e-12)
    shifted = jnp.concatenate(
        [gnorm[1:], jnp.zeros((1, _G), jnp.float32)], axis=0)
    sim = jnp.sum(gnorm * shifted, axis=1, keepdims=True)
    pid = lax.broadcasted_iota(jnp.int32, (_S, 1), 0)
    sim = jnp.where(pid >= _S - 1, jnp.float32(-2.0), sim)
    simrow_ref[...] = jnp.reshape(sim, (1, _S))
    qi0 = lax.broadcasted_iota(jnp.int32, (_S, 128), 1)

    def chunk(c, acc):
        qs = simrow_ref[:, pl.ds(c * 128, 128)]
        qi = qi0 + c * 128
        gt = qs > sim
        eq = (qs == sim) & (qi < pid)
        return acc + (gt | eq).astype(jnp.int32)

    acc = lax.fori_loop(0, _S // 128, chunk, jnp.zeros((_S, 128), jnp.int32))
    rank = jnp.sum(acc, axis=1, keepdims=True)
    rank_ref[...] = jnp.reshape(rank, (1, 1, _S))
    norm_ref[...] = jnp.reshape(gn, (1, 1, _S))


def _scores(x, w):
    return pl.pallas_call(
        _score_body,
        grid=(_B,),
        in_specs=[pl.BlockSpec((1, _S, _D), lambda b: (b, 0, 0)),
                  pl.BlockSpec((_G, _D), lambda b: (0, 0))],
        out_specs=[pl.BlockSpec((1, 1, _S), lambda b: (b, 0, 0)),
                   pl.BlockSpec((1, 1, _S), lambda b: (b, 0, 0))],
        out_shape=[jax.ShapeDtypeStruct((_B, 1, _S), jnp.int32),
                   jax.ShapeDtypeStruct((_B, 1, _S), jnp.float32)],
        scratch_shapes=[pltpu.VMEM((1, _S), jnp.float32)],
    )(x, w)


def _greedy_body(rank_hbm, norm_hbm, pos_hbm,
                 ui_hbm, up_hbm, mi_hbm, mj_hbm, mp_hbm, c0_hbm, c1_hbm,
                 po_hbm,
                 rank_v, order_v, used_v, isi_v, norms_v, posrow_v,
                 ui_v, up_v, mi_v, mj_v, mp_v, c0_v, c1_v, pos_v):
    wid = lax.axis_index("s") * _NC + lax.axis_index("c")

    @pl.when(wid < _B)
    def _():
        b = wid
        pltpu.sync_copy(rank_hbm.at[b], rank_v)
        pltpu.sync_copy(norm_hbm.at[b], norms_v.at[pl.ds(0, _S)])
        pltpu.sync_copy(pos_hbm.at[b], posrow_v)
        norms_v[pl.ds(_S, 16)] = jnp.zeros((16,), jnp.float32)

        zeros16 = jnp.zeros((16,), jnp.int32)

        def init_chunk(ci, _):
            b16 = ci * 16
            used_v[pl.ds(b16, 16)] = zeros16
            isi_v[pl.ds(b16, 16)] = zeros16
            vals = lax.iota(jnp.int32, 16) + b16
            idx = rank_v[pl.ds(b16, 16)]
            plsc.store_scatter(order_v, [idx], vals)
            return 0

        lax.fori_loop(0, _S // 16, init_chunk, 0)

        lanes = lax.iota(jnp.int32, 16)
        ones16 = jnp.ones((16,), jnp.int32)

        def g_cond(carry):
            t, cnt = carry
            return jnp.logical_and(t < _S - 1, cnt < _R)

        def g_body(carry):
            t, cnt = carry
            p = order_v[pl.ds(t, 16)][0]
            u2 = used_v[pl.ds(p, 16)]
            free = (u2[0] + u2[1]) == 0

            @pl.when(free)
            def _():
                plsc.store_scatter(used_v, [p + lanes], ones16,
                                   mask=lanes < 2)
                plsc.store_scatter(isi_v, [p + lanes], ones16,
                                   mask=lanes < 1)

            return (t + 1, cnt + free.astype(jnp.int32))

        lax.while_loop(g_cond, g_body, (jnp.int32(0), jnp.int32(0)))

        def comp_chunk(ci, carry):
            base, ubase, mbase = carry
            b16 = ci * 16
            tvec = lax.iota(jnp.int32, 16) + b16
            usedc = used_v[pl.ds(b16, 16)]
            isic = isi_v[pl.ds(b16, 16)]
            isb = isic == 1
            keep = jnp.logical_or(usedc == 0, isb)
            kint = keep.astype(jnp.int32)
            posv = base + plsc.cumsum(kint) - kint
            ukeep = jnp.logical_and(keep, jnp.logical_not(isb))
            uint = ukeep.astype(jnp.int32)
            uposv = ubase + plsc.cumsum(uint) - uint
            mint = isic
            mposv = mbase + plsc.cumsum(mint) - mint
            n0 = norms_v[pl.ds(b16, 16)]
            n1 = norms_v[pl.ds(b16 + 1, 16)]
            den = n0 + n1 + jnp.float32(1e-8)
            gin = b * _S
            gout = b * _KEEP
            plsc.store_scatter(ui_v, [uposv], tvec + gin, mask=ukeep)
            plsc.store_scatter(up_v, [uposv], posv + gout, mask=ukeep)
            plsc.store_scatter(mi_v, [mposv], tvec + gin, mask=isb)
            plsc.store_scatter(mj_v, [mposv], tvec + gin + 1, mask=isb)
            plsc.store_scatter(mp_v, [mposv], posv + gout, mask=isb)
            plsc.store_scatter(c0_v, [mposv], n0 / den, mask=isb)
            plsc.store_scatter(c1_v, [mposv], n1 / den, mask=isb)
            plsc.store_scatter(pos_v, [posv], posrow_v[pl.ds(b16, 16)],
                               mask=keep)
            return (base + jnp.sum(kint), ubase + jnp.sum(uint),
                    mbase + jnp.sum(mint))

        lax.fori_loop(0, _S // 16, comp_chunk,
                      (jnp.int32(0), jnp.int32(0), jnp.int32(0)))

        pltpu.sync_copy(ui_v, ui_hbm.at[b])
        pltpu.sync_copy(up_v, up_hbm.at[b])
        pltpu.sync_copy(mi_v, mi_hbm.at[b])
        pltpu.sync_copy(mj_v, mj_hbm.at[b])
        pltpu.sync_copy(mp_v, mp_hbm.at[b])
        pltpu.sync_copy(c0_v, c0_hbm.at[b])
        pltpu.sync_copy(c1_v, c1_hbm.at[b])
        pltpu.sync_copy(pos_v, po_hbm.at[b])


def _greedy(rank, norm, position_ids):
    mesh = plsc.VectorSubcoreMesh(core_axis_name="c", subcore_axis_name="s",
                                  num_cores=_NC, num_subcores=_NS)
    f = pl.kernel(
        _greedy_body,
        out_type=[jax.ShapeDtypeStruct((_B, _UN), jnp.int32),
                  jax.ShapeDtypeStruct((_B, _UN), jnp.int32),
                  jax.ShapeDtypeStruct((_B, _R), jnp.int32),
                  jax.ShapeDtypeStruct((_B, _R), jnp.int32),
                  jax.ShapeDtypeStruct((_B, _R), jnp.int32),
                  jax.ShapeDtypeStruct((_B, _R), jnp.float32),
                  jax.ShapeDtypeStruct((_B, _R), jnp.float32),
                  jax.ShapeDtypeStruct((_B, _KEEP), jnp.int32)],
        mesh=mesh,
        scratch_types=[pltpu.VMEM((_S,), jnp.int32),
                       pltpu.VMEM((_S + 16,), jnp.int32),
                       pltpu.VMEM((_S + 16,), jnp.int32),
                       pltpu.VMEM((_S + 16,), jnp.int32),
                       pltpu.VMEM((_S + 16,), jnp.float32),
                       pltpu.VMEM((_S,), jnp.int32),
                       pltpu.VMEM((_UN,), jnp.int32),
                       pltpu.VMEM((_UN,), jnp.int32),
                       pltpu.VMEM((_R,), jnp.int32),
                       pltpu.VMEM((_R,), jnp.int32),
                       pltpu.VMEM((_R,), jnp.int32),
                       pltpu.VMEM((_R,), jnp.float32),
                       pltpu.VMEM((_R,), jnp.float32),
                       pltpu.VMEM((_KEEP,), jnp.int32)],
        compiler_params=pltpu.CompilerParams(needs_layout_passes=False),
    )
    return f(rank, norm, position_ids)


def _merge_body(x_hbm, s_hbm, ui_hbm, up2_hbm, mi_hbm, mj_hbm, mp2_hbm,
                c0_hbm, c1_hbm,
                xo_hbm, so_hbm,
                ui_v, mi_v, mj_v, c0_v, c1_v, op2_v, mp2_v,
                xa0, xa1, xb0, xb1, sa0, sa1, sb0, sb1,
                sgx0, sgx1, sgxb0, sgxb1, sgs0, sgs1, sgsb0, sgsb1,
                ssx0, ssx1, sss0, sss1):
    xa, xb, sa, sb = [xa0, xa1], [xb0, xb1], [sa0, sa1], [sb0, sb1]
    sgx, sgxb = [sgx0, sgx1], [sgxb0, sgxb1]
    sgs, sgsb = [sgs0, sgs1], [sgsb0, sgsb1]
    ssx, sss = [ssx0, ssx1], [sss0, sss1]

    wid = lax.axis_index("s") * _NC + lax.axis_index("c")
    ub = wid * _UPW
    mb = wid * _MPW
    pltpu.sync_copy(ui_hbm.at[pl.ds(ub, _UPW)], ui_v)
    pltpu.sync_copy(up2_hbm.at[pl.ds(wid * _NU, _NU)], op2_v)
    pltpu.sync_copy(mi_hbm.at[pl.ds(mb, _MPW)], mi_v)
    pltpu.sync_copy(mj_hbm.at[pl.ds(mb, _MPW)], mj_v)
    pltpu.sync_copy(mp2_hbm.at[pl.ds(wid * _NM, _NM)], mp2_v)
    pltpu.sync_copy(c0_hbm.at[pl.ds(mb, _MPW)], c0_v.at[pl.ds(0, _MPW)])
    pltpu.sync_copy(c1_hbm.at[pl.ds(mb, _MPW)], c1_v.at[pl.ds(0, _MPW)])

    hgx = [None, None]
    hgs = [None, None]
    hgxb = [None, None]
    hgsb = [None, None]
    hsx = [None, None]
    hss = [None, None]

    def issue1(c):
        i = c & 1
        ia = ui_v.at[pl.ds(c * _CH, _CH)]
        hgx[i] = pltpu.async_copy(x_hbm.at[ia], xa[i], sgx[i])
        hgs[i] = pltpu.async_copy(s_hbm.at[ia], sa[i], sgs[i])

    def process1(c):
        i = c & 1
        op = op2_v.at[c]
        hgx[i].wait()
        hsx[i] = pltpu.async_copy(xa[i], xo_hbm.at[op], ssx[i])
        hgs[i].wait()
        hss[i] = pltpu.async_copy(sa[i], so_hbm.at[op], sss[i])

    issue1(0)
    for c in range(1, _NU):
        i = c & 1
        if hsx[i] is not None:
            hsx[i].wait()
            hss[i].wait()
        issue1(c)
        process1(c - 1)
    process1(_NU - 1)
    for i in (0, 1):
        hsx[i].wait()
        hss[i].wait()
        hsx[i] = None
        hss[i] = None

    def issue2(c):
        i = c & 1
        ia = mi_v.at[pl.ds(c * _CH, _CH)]
        ib = mj_v.at[pl.ds(c * _CH, _CH)]
        hgx[i] = pltpu.async_copy(x_hbm.at[ia], xa[i], sgx[i])
        hgxb[i] = pltpu.async_copy(x_hbm.at[ib], xb[i], sgxb[i])
        hgs[i] = pltpu.async_copy(s_hbm.at[ia], sa[i], sgs[i])
        hgsb[i] = pltpu.async_copy(s_hbm.at[ib], sb[i], sgsb[i])

    def process2(c):
        i = c & 1
        cb = c * _CH
        op = mp2_v.at[c]
        hgx[i].wait()
        hgxb[i].wait()
        xai, xbi = xa[i], xb[i]
        sai, sbi = sa[i], sb[i]

        def xrow(o, _):
            c0s = c0_v[pl.ds(cb + o, 16)][0]
            c1s = c1_v[pl.ds(cb + o, 16)][0]

            def xlane(d, _):
                sl = pl.ds(d * 16, 16)
                xai[o, sl] = c0s * xai[o, sl] + c1s * xbi[o, sl]
                return 0

            lax.fori_loop(0, _D // 16, xlane, 0)
            return 0

        lax.fori_loop(0, _CH, xrow, 0)
        hsx[i] = pltpu.async_copy(xai, xo_hbm.at[op], ssx[i])

        hgs[i].wait()
        hgsb[i].wait()

        def srow(o, _):
            def slane(d, _):
                sl = pl.ds(d * 16, 16)
                sai[o, sl] = sai[o, sl] + sbi[o, sl]
                return 0

            lax.fori_loop(0, _S // 16, slane, 0)
            return 0

        lax.fori_loop(0, _CH, srow, 0)
        hss[i] = pltpu.async_copy(sai, so_hbm.at[op], sss[i])

    issue2(0)
    for c in range(1, _NM):
        i = c & 1
        if hsx[i] is not None:
            hsx[i].wait()
            hss[i].wait()
        issue2(c)
        process2(c - 1)
    process2(_NM - 1)
    for i in (0, 1):
        hsx[i].wait()
        hss[i].wait()


def _merge(x2, s2, ui, up2, mi, mj, mp2, c0, c1):
    mesh = plsc.VectorSubcoreMesh(core_axis_name="c", subcore_axis_name="s",
                                  num_cores=_NC, num_subcores=_NS)
    f = pl.kernel(
        _merge_body,
        out_type=[jax.ShapeDtypeStruct((_TOT, _D), jnp.float32),
                  jax.ShapeDtypeStruct((_TOT, _S), jnp.float32)],
        mesh=mesh,
        scratch_types=[pltpu.VMEM((_UPW,), jnp.int32),
                       pltpu.VMEM((_MPW,), jnp.int32),
                       pltpu.VMEM((_MPW,), jnp.int32),
                       pltpu.VMEM((_MPW + 16,), jnp.float32),
                       pltpu.VMEM((_MPW + 16,), jnp.float32),
                       pltpu.VMEM((_NU, _CH), jnp.int32),
                       pltpu.VMEM((_NM, _CH), jnp.int32),
                       pltpu.VMEM((_CH, _D), jnp.float32),
                       pltpu.VMEM((_CH, _D), jnp.float32),
                       pltpu.VMEM((_CH, _D), jnp.float32),
                       pltpu.VMEM((_CH, _D), jnp.float32),
                       pltpu.VMEM((_CH, _S), jnp.float32),
                       pltpu.VMEM((_CH, _S), jnp.float32),
                       pltpu.VMEM((_CH, _S), jnp.float32),
                       pltpu.VMEM((_CH, _S), jnp.float32)]
                      + [pltpu.SemaphoreType.DMA] * 12,
        compiler_params=pltpu.CompilerParams(needs_layout_passes=False),
    )
    return f(x2, s2, ui, up2, mi, mj, mp2, c0, c1)


def kernel(x, source, position_ids, W, r, window_size):
    anchor = (jnp.asarray(r) - _R) + (jnp.asarray(window_size) - 2)
    x = x + anchor.astype(x.dtype) * 0
    rank, norm = _scores(x, W)
    rank = rank.reshape(_B, _S)
    norm = norm.reshape(_B, _S)
    ui, up, mi, mj, mp, c0, c1, pos_out = _greedy(rank, norm, position_ids)
    x2 = x.reshape(_B * _S, _D)
    s2 = source.reshape(_B * _S, _S)
    xo, so = _merge(x2, s2, ui.reshape(_UTOT), up.reshape(_UTOT // _CH, _CH),
                    mi.reshape(_MTOT), mj.reshape(_MTOT),
                    mp.reshape(_MTOT // _CH, _CH),
                    c0.reshape(_MTOT), c1.reshape(_MTOT))
    return (xo.reshape(_B, _KEEP, _D), so.reshape(_B, _KEEP, _S), pos_out)


# speedup vs baseline: 34.1481x; 1.0364x over previous
import jax
import jax.numpy as jnp
from jax import lax
from jax.experimental import pallas as pl
from jax.experimental.pallas import tpu as pltpu
from jax.experimental.pallas import tpu_sc as plsc

_B, _S, _D = 4, 2048, 768
_G = 64
_R = 512
_KEEP = _S - _R
_NC, _NS = 2, 16
_TOT = _B * _KEEP
_UN = _KEEP - _R
_UTOT = _B * _UN
_MTOT = _B * _R
_CH = 8
_UPT = 2 * (_UN // _NS)
_MPT = 2 * (_R // _NS)
_NU = _UPT // _CH
_NM = _MPT // _CH


def _score_body(x_ref, w_ref, rank_ref, norm_ref, simrow_ref):
    xb = x_ref[0]
    w = w_ref[...]
    g = lax.dot_general(xb, w, (((1,), (1,)), ((), ())),
                        preferred_element_type=jnp.float32)
    gn = jnp.sqrt(jnp.sum(g * g, axis=1, keepdims=True))
    gnorm = g / jnp.maximum(gn, 1e-12)
    shifted = jnp.concatenate(
        [gnorm[1:], jnp.zeros((1, _G), jnp.float32)], axis=0)
    sim = jnp.sum(gnorm * shifted, axis=1, keepdims=True)
    pid = lax.broadcasted_iota(jnp.int32, (_S, 1), 0)
    sim = jnp.where(pid >= _S - 1, jnp.float32(-2.0), sim)
    simrow_ref[...] = jnp.reshape(sim, (1, _S))
    qi0 = lax.broadcasted_iota(jnp.int32, (_S, 128), 1)

    def chunk(c, acc):
        qs = simrow_ref[:, pl.ds(c * 128, 128)]
        qi = qi0 + c * 128
        gt = qs > sim
        eq = (qs == sim) & (qi < pid)
        return acc + (gt | eq).astype(jnp.int32)

    acc = lax.fori_loop(0, _S // 128, chunk, jnp.zeros((_S, 128), jnp.int32))
    rank = jnp.sum(acc, axis=1, keepdims=True)
    rank_ref[...] = jnp.reshape(rank, (1, 1, _S))
    norm_ref[...] = jnp.reshape(gn, (1, 1, _S))


def _scores(x, w):
    return pl.pallas_call(
        _score_body,
        grid=(_B,),
        in_specs=[pl.BlockSpec((1, _S, _D), lambda b: (b, 0, 0)),
                  pl.BlockSpec((_G, _D), lambda b: (0, 0))],
        out_specs=[pl.BlockSpec((1, 1, _S), lambda b: (b, 0, 0)),
                   pl.BlockSpec((1, 1, _S), lambda b: (b, 0, 0))],
        out_shape=[jax.ShapeDtypeStruct((_B, 1, _S), jnp.int32),
                   jax.ShapeDtypeStruct((_B, 1, _S), jnp.float32)],
        scratch_shapes=[pltpu.VMEM((1, _S), jnp.float32)],
    )(x, w)


def _fused_body(rank_hbm, norm_hbm, pos_hbm, x_hbm, s_hbm,
                xo_hbm, so_hbm, po_hbm,
                ui_hbm, up2_hbm, mi_hbm, mj_hbm, mp2_hbm, c0_hbm, c1_hbm,
                rank_v, order_v, used_v, isi_v, norms_v, posrow_v,
                ui_v, up_v, mi_v, mj_v, mp_v, c0b_v, c1b_v, pos_v,
                ui3_v, op23_v, mi3_v, mj3_v, mp23_v, c03_v, c13_v,
                xa0, xa1, xb0, xb1, sa0, sa1, sb0,
                sgx0, sgx1, sgxb0, sgxb1, sgs0, sgs1, sgsb0, sgsb1,
                ssx0, ssx1, sss0, sss1):
    s_idx = lax.axis_index("s")
    c_idx = lax.axis_index("c")

    @pl.when(s_idx < 2)
    def _():
        b = c_idx + 2 * s_idx
        pltpu.sync_copy(rank_hbm.at[b], rank_v)
        pltpu.sync_copy(norm_hbm.at[b], norms_v.at[pl.ds(0, _S)])
        pltpu.sync_copy(pos_hbm.at[b], posrow_v)
        norms_v[pl.ds(_S, 16)] = jnp.zeros((16,), jnp.float32)

        zeros16 = jnp.zeros((16,), jnp.int32)

        def init_chunk(ci, _):
            b16 = ci * 16
            used_v[pl.ds(b16, 16)] = zeros16
            isi_v[pl.ds(b16, 16)] = zeros16
            vals = lax.iota(jnp.int32, 16) + b16
            idx = rank_v[pl.ds(b16, 16)]
            plsc.store_scatter(order_v, [idx], vals)
            return 0

        lax.fori_loop(0, _S // 16, init_chunk, 0)

        lanes = lax.iota(jnp.int32, 16)
        ones16 = jnp.ones((16,), jnp.int32)

        def g_cond(carry):
            t, cnt = carry
            return jnp.logical_and(t < _S - 1, cnt < _R)

        def g_body(carry):
            t, cnt = carry
            p = order_v[pl.ds(t, 16)][0]
            u2 = used_v[pl.ds(p, 16)]
            free = (u2[0] + u2[1]) == 0

            @pl.when(free)
            def _():
                plsc.store_scatter(used_v, [p + lanes], ones16,
                                   mask=lanes < 2)
                plsc.store_scatter(isi_v, [p + lanes], ones16,
                                   mask=lanes < 1)

            return (t + 1, cnt + free.astype(jnp.int32))

        lax.while_loop(g_cond, g_body, (jnp.int32(0), jnp.int32(0)))

        def comp_chunk(ci, carry):
            base, ubase, mbase = carry
            b16 = ci * 16
            tvec = lax.iota(jnp.int32, 16) + b16
            usedc = used_v[pl.ds(b16, 16)]
            isic = isi_v[pl.ds(b16, 16)]
            isb = isic == 1
            keep = jnp.logical_or(usedc == 0, isb)
            kint = keep.astype(jnp.int32)
            posv = base + plsc.cumsum(kint) - kint
            ukeep = jnp.logical_and(keep, jnp.logical_not(isb))
            uint = ukeep.astype(jnp.int32)
            uposv = ubase + plsc.cumsum(uint) - uint
            mint = isic
            mposv = mbase + plsc.cumsum(mint) - mint
            n0 = norms_v[pl.ds(b16, 16)]
            n1 = norms_v[pl.ds(b16 + 1, 16)]
            den = n0 + n1 + jnp.float32(1e-8)
            gin = b * _S
            gout = b * _KEEP
            plsc.store_scatter(ui_v, [uposv], tvec + gin, mask=ukeep)
            plsc.store_scatter(up_v, [uposv >> 3, uposv & 7], posv + gout,
                               mask=ukeep)
            plsc.store_scatter(mi_v, [mposv], tvec + gin, mask=isb)
            plsc.store_scatter(mj_v, [mposv], tvec + gin + 1, mask=isb)
            plsc.store_scatter(mp_v, [mposv >> 3, mposv & 7], posv + gout,
                               mask=isb)
            plsc.store_scatter(c0b_v, [mposv], n0 / den, mask=isb)
            plsc.store_scatter(c1b_v, [mposv], n1 / den, mask=isb)
            plsc.store_scatter(pos_v, [posv], posrow_v[pl.ds(b16, 16)],
                               mask=keep)
            return (base + jnp.sum(kint), ubase + jnp.sum(uint),
                    mbase + jnp.sum(mint))

        lax.fori_loop(0, _S // 16, comp_chunk,
                      (jnp.int32(0), jnp.int32(0), jnp.int32(0)))

        pltpu.sync_copy(ui_v, ui_hbm.at[pl.ds(b * _UN, _UN)])
        pltpu.sync_copy(up_v, up2_hbm.at[pl.ds(b * (_UN // 8), _UN // 8)])
        pltpu.sync_copy(mi_v, mi_hbm.at[pl.ds(b * _R, _R)])
        pltpu.sync_copy(mj_v, mj_hbm.at[pl.ds(b * _R, _R)])
        pltpu.sync_copy(mp_v, mp2_hbm.at[pl.ds(b * (_R // 8), _R // 8)])
        pltpu.sync_copy(c0b_v, c0_hbm.at[pl.ds(b * _R, _R)])
        pltpu.sync_copy(c1b_v, c1_hbm.at[pl.ds(b * _R, _R)])
        pltpu.sync_copy(pos_v, po_hbm.at[b])

    plsc.subcore_barrier()

    upb = _UN // _NS
    mpb = _R // _NS
    for seg in range(2):
        bb = c_idx + 2 * seg
        pltpu.sync_copy(ui_hbm.at[pl.ds(bb * _UN + s_idx * upb, upb)],
                        ui3_v.at[pl.ds(seg * upb, upb)])
        pltpu.sync_copy(
            up2_hbm.at[pl.ds(bb * (_UN // 8) + s_idx * (upb // 8), upb // 8)],
            op23_v.at[pl.ds(seg * (upb // 8), upb // 8)])
        pltpu.sync_copy(mi_hbm.at[pl.ds(bb * _R + s_idx * mpb, mpb)],
                        mi3_v.at[pl.ds(seg * mpb, mpb)])
        pltpu.sync_copy(mj_hbm.at[pl.ds(bb * _R + s_idx * mpb, mpb)],
                        mj3_v.at[pl.ds(seg * mpb, mpb)])
        pltpu.sync_copy(
            mp2_hbm.at[pl.ds(bb * (_R // 8) + s_idx * (mpb // 8), mpb // 8)],
            mp23_v.at[pl.ds(seg * (mpb // 8), mpb // 8)])
        pltpu.sync_copy(c0_hbm.at[pl.ds(bb * _R + s_idx * mpb, mpb)],
                        c03_v.at[pl.ds(seg * mpb, mpb)])
        pltpu.sync_copy(c1_hbm.at[pl.ds(bb * _R + s_idx * mpb, mpb)],
                        c13_v.at[pl.ds(seg * mpb, mpb)])

    xa, xb, sa, sb = [xa0, xa1], [xb0, xb1], [sa0, sa1], [sb0, sb0]
    sgx, sgxb = [sgx0, sgx1], [sgxb0, sgxb1]
    sgs, sgsb = [sgs0, sgs1], [sgsb0, sgsb1]
    ssx, sss = [ssx0, ssx1], [sss0, sss1]

    hgx = [None, None]
    hgs = [None, None]
    hgxb = [None, None]
    hgsb = [None, None]
    hsx = [None, None]
    hss = [None, None]

    def issue1(c):
        i = c & 1
        ia = ui3_v.at[pl.ds(c * _CH, _CH)]
        hgx[i] = pltpu.async_copy(x_hbm.at[ia], xa[i], sgx[i])
        hgs[i] = pltpu.async_copy(s_hbm.at[ia], sa[i], sgs[i])

    def process1(c):
        i = c & 1
        op = op23_v.at[c]
        hgx[i].wait()
        hsx[i] = pltpu.async_copy(xa[i], xo_hbm.at[op], ssx[i])
        hgs[i].wait()
        hss[i] = pltpu.async_copy(sa[i], so_hbm.at[op], sss[i])

    issue1(0)
    for c in range(1, _NU):
        i = c & 1
        if hsx[i] is not None:
            hsx[i].wait()
            hss[i].wait()
        issue1(c)
        process1(c - 1)
    process1(_NU - 1)
    for i in (0, 1):
        hsx[i].wait()
        hss[i].wait()
        hsx[i] = None
        hss[i] = None

    def issue2(c):
        i = c & 1
        ia = mi3_v.at[pl.ds(c * _CH, _CH)]
        ib = mj3_v.at[pl.ds(c * _CH, _CH)]
        hgx[i] = pltpu.async_copy(x_hbm.at[ia], xa[i], sgx[i])
        hgxb[i] = pltpu.async_copy(x_hbm.at[ib], xb[i], sgxb[i])
        hgs[i] = pltpu.async_copy(s_hbm.at[ia], sa[i], sgs[i])

    def issue2b(c):
        i = c & 1
        ib = mj3_v.at[pl.ds(c * _CH, _CH)]
        hgsb[i] = pltpu.async_copy(s_hbm.at[ib], sb[i], sgsb[i])

    def process2(c):
        i = c & 1
        cb = c * _CH
        op = mp23_v.at[c]
        hgx[i].wait()
        hgxb[i].wait()
        xai, xbi = xa[i], xb[i]
        sai, sbi = sa[i], sb[i]

        def xrow(o, _):
            c0s = c03_v[pl.ds(cb + o, 16)][0]
            c1s = c13_v[pl.ds(cb + o, 16)][0]

            def xlane(d, _):
                sl = pl.ds(d * 16, 16)
                xai[o, sl] = c0s * xai[o, sl] + c1s * xbi[o, sl]
                return 0

            lax.fori_loop(0, _D // 16, xlane, 0)
            return 0

        lax.fori_loop(0, _CH, xrow, 0)
        hsx[i] = pltpu.async_copy(xai, xo_hbm.at[op], ssx[i])

        hgs[i].wait()
        hgsb[c & 1].wait()

        def srow(o, _):
            def slane(d, _):
                sl = pl.ds(d * 16, 16)
                sai[o, sl] = sai[o, sl] + sbi[o, sl]
                return 0

            lax.fori_loop(0, _S // 16, slane, 0)
            return 0

        lax.fori_loop(0, _CH, srow, 0)
        hss[i] = pltpu.async_copy(sai, so_hbm.at[op], sss[i])
        if c + 1 < _NM:
            issue2b(c + 1)

    issue2(0)
    issue2b(0)
    for c in range(1, _NM):
        i = c & 1
        if hsx[i] is not None:
            hsx[i].wait()
            hss[i].wait()
        issue2(c)
        process2(c - 1)
    process2(_NM - 1)
    for i in (0, 1):
        hsx[i].wait()
        hss[i].wait()


def _fused(rank, norm, position_ids, x2, s2):
    mesh = plsc.VectorSubcoreMesh(core_axis_name="c", subcore_axis_name="s",
                                  num_cores=_NC, num_subcores=_NS)
    f = pl.kernel(
        _fused_body,
        out_type=[jax.ShapeDtypeStruct((_TOT, _D), jnp.float32),
                  jax.ShapeDtypeStruct((_TOT, _S), jnp.float32),
                  jax.ShapeDtypeStruct((_B, _KEEP), jnp.int32),
                  jax.ShapeDtypeStruct((_UTOT,), jnp.int32),
                  jax.ShapeDtypeStruct((_UTOT // 8, 8), jnp.int32),
                  jax.ShapeDtypeStruct((_MTOT,), jnp.int32),
                  jax.ShapeDtypeStruct((_MTOT,), jnp.int32),
                  jax.ShapeDtypeStruct((_MTOT // 8, 8), jnp.int32),
                  jax.ShapeDtypeStruct((_MTOT,), jnp.float32),
                  jax.ShapeDtypeStruct((_MTOT,), jnp.float32)],
        mesh=mesh,
        scratch_types=[pltpu.VMEM((_S,), jnp.int32),
                       pltpu.VMEM((_S + 16,), jnp.int32),
                       pltpu.VMEM((_S + 16,), jnp.int32),
                       pltpu.VMEM((_S + 16,), jnp.int32),
                       pltpu.VMEM((_S + 16,), jnp.float32),
                       pltpu.VMEM((_S,), jnp.int32),
                       pltpu.VMEM((_UN,), jnp.int32),
                       pltpu.VMEM((_UN // 8, 8), jnp.int32),
                       pltpu.VMEM((_R,), jnp.int32),
                       pltpu.VMEM((_R,), jnp.int32),
                       pltpu.VMEM((_R // 8, 8), jnp.int32),
                       pltpu.VMEM((_R,), jnp.float32),
                       pltpu.VMEM((_R,), jnp.float32),
                       pltpu.VMEM((_KEEP,), jnp.int32),
                       pltpu.VMEM((_UPT,), jnp.int32),
                       pltpu.VMEM((_NU, _CH), jnp.int32),
                       pltpu.VMEM((_MPT,), jnp.int32),
                       pltpu.VMEM((_MPT,), jnp.int32),
                       pltpu.VMEM((_NM, _CH), jnp.int32),
                       pltpu.VMEM((_MPT + 16,), jnp.float32),
                       pltpu.VMEM((_MPT + 16,), jnp.float32),
                       pltpu.VMEM((_CH, _D), jnp.float32),
                       pltpu.VMEM((_CH, _D), jnp.float32),
                       pltpu.VMEM((_CH, _D), jnp.float32),
                       pltpu.VMEM((_CH, _D), jnp.float32),
                       pltpu.VMEM((_CH, _S), jnp.float32),
                       pltpu.VMEM((_CH, _S), jnp.float32),
                       pltpu.VMEM((_CH, _S), jnp.float32)]
                      + [pltpu.SemaphoreType.DMA] * 12,
        compiler_params=pltpu.CompilerParams(needs_layout_passes=False),
    )
    return f(rank, norm, position_ids, x2, s2)


def kernel(x, source, position_ids, W, r, window_size):
    anchor = (jnp.asarray(r) - _R) + (jnp.asarray(window_size) - 2)
    x = x + anchor.astype(x.dtype) * 0
    rank, norm = _scores(x, W)
    rank = rank.reshape(_B, _S)
    norm = norm.reshape(_B, _S)
    x2 = x.reshape(_B * _S, _D)
    s2 = source.reshape(_B * _S, _S)
    outs = _fused(rank, norm, position_ids, x2, s2)
    xo, so, pos_out = outs[0], outs[1], outs[2]
    return (xo.reshape(_B, _KEEP, _D), so.reshape(_B, _KEEP, _S), pos_out)
